# Initial kernel scaffold; baseline (speedup 1.0000x reference)
import jax
import jax.numpy as jnp
from jax.experimental import pallas as pl


def kernel(x, weight):
    raise NotImplementedError("write your pallas kernel here")


# speedup vs baseline: 3.4425x; 3.4425x over previous
import jax
import jax.numpy as jnp
from jax import lax
from jax.experimental import pallas as pl
from jax.experimental.pallas import tpu as pltpu
from jax.experimental.pallas import tpu_sc as plsc

DIM = 64
PAD_DIM = 128
WINDOW = 128
NC = 2
NS = 16
NW = NC * NS


def _gather_kernel(num_indices):
    chunks = num_indices // WINDOW
    cpw = chunks // NW
    mesh = plsc.VectorSubcoreMesh(core_axis_name="c", subcore_axis_name="s")

    @pl.kernel(
        out_type=jax.ShapeDtypeStruct((num_indices, PAD_DIM), jnp.float32),
        mesh=mesh,
        scratch_types=[
            pltpu.VMEM((cpw, WINDOW), jnp.int32),
            pltpu.VMEM((WINDOW, PAD_DIM), jnp.float32),
            pltpu.SemaphoreType.DMA,
        ],
    )
    def kern(table_hbm, idx_hbm, out_hbm, idx_v, rows_v, sem):
        wid = lax.axis_index("s") * NC + lax.axis_index("c")
        pltpu.sync_copy(idx_hbm.at[wid], idx_v)

        @pl.loop(0, cpw)
        def _(j):
            pltpu.async_copy(table_hbm.at[idx_v.at[j]], rows_v, sem).wait()
            base = (wid * cpw + j) * WINDOW
            pltpu.sync_copy(rows_v, out_hbm.at[pl.ds(base, WINDOW)])

    return kern


def kernel(x, weight):
    b, s = x.shape
    n = b * s
    idx = x.reshape(NW, n // (NW * WINDOW), WINDOW).astype(jnp.int32)
    table = jnp.pad(weight, ((0, 0), (0, PAD_DIM - DIM)))
    out = _gather_kernel(n)(table, idx)
    return out[:, :DIM].reshape(b, s, DIM)
